# baseline (speedup 1.0000x reference)
import functools

import jax
import jax.numpy as jnp
from jax import lax
from jax.experimental import pallas as pl
from jax.experimental.pallas import tpu as pltpu
from jax.experimental.pallas import tpu_sc as plsc

_N = 2048
_D = 768
_E = 8
_F = 3072
_BLK = 128
_NBLK = 2 * _N // _BLK + _E
_NPAD = _NBLK * _BLK
_CH = 512
_NW = 32


def _prep_body(x_ref, g_ref, b_ref, wg_ref,
               xn_ref, dest_ref, tp_ref, sp_ref, aux_ref):
    xv = x_ref[...]
    mu = jnp.mean(xv, axis=1, keepdims=True)
    xc = xv - mu
    var = jnp.mean(xc * xc, axis=1, keepdims=True)
    xn = xc * lax.rsqrt(var + 1e-5) * g_ref[...] + b_ref[...]
    xn_ref[...] = xn

    logits = lax.dot_general(xn, wg_ref[...], (((1,), (1,)), ((), ())),
                             preferred_element_type=jnp.float32)
    z = logits - jnp.max(logits, axis=1, keepdims=True)
    ez = jnp.exp(z)
    probs = ez / jnp.sum(ez, axis=1, keepdims=True)

    idx = lax.broadcasted_iota(jnp.int32, (_N, _E), 1)
    m1 = jnp.max(probs, axis=1, keepdims=True)
    i1 = jnp.min(jnp.where(probs == m1, idx, _E), axis=1, keepdims=True)
    masked = jnp.where(idx == i1, -1.0, probs)
    m2 = jnp.max(masked, axis=1, keepdims=True)
    i2 = jnp.min(jnp.where(masked == m2, idx, _E), axis=1, keepdims=True)
    s = m1 + m2
    tp_ref[0:_N, :] = m1 / s
    tp_ref[_N:2 * _N, :] = m2 / s

    oh0 = (idx == i1).astype(jnp.float32)
    oh1 = (idx == i2).astype(jnp.float32)

    tri = (lax.broadcasted_iota(jnp.int32, (_CH, _CH), 0)
           > lax.broadcasted_iota(jnp.int32, (_CH, _CH), 1)).astype(jnp.float32)
    carry = jnp.zeros((1, _E), jnp.float32)
    ranks = []
    chunks = []
    for c in range(2 * _N // _CH):
        src = oh0 if c < _N // _CH else oh1
        r0 = (c % (_N // _CH)) * _CH
        ohc = src[r0:r0 + _CH, :]
        rm = lax.dot_general(tri, ohc, (((1,), (0,)), ((), ())),
                             preferred_element_type=jnp.float32) + carry
        ranks.append(jnp.sum(rm * ohc, axis=1, keepdims=True))
        chunks.append(ohc)
        carry = carry + jnp.sum(ohc, axis=0, keepdims=True)
    counts = carry

    pc = jnp.ceil(counts / _BLK) * _BLK
    lt = (lax.broadcasted_iota(jnp.int32, (_E, _E), 0)
          < lax.broadcasted_iota(jnp.int32, (_E, _E), 1)).astype(jnp.float32)
    le = (lax.broadcasted_iota(jnp.int32, (_E, _E), 0)
          <= lax.broadcasted_iota(jnp.int32, (_E, _E), 1)).astype(jnp.float32)
    po = lax.dot_general(pc, lt, (((1,), (0,)), ((), ())),
                         preferred_element_type=jnp.float32)
    cum = lax.dot_general(pc, le, (((1,), (0,)), ((), ())),
                          preferred_element_type=jnp.float32)

    for c in range(2 * _N // _CH):
        ohc = chunks[c]
        posel = jnp.sum(po * ohc, axis=1, keepdims=True)
        d = (ranks[c] + posel).astype(jnp.int32)
        dest_ref[c * _CH:(c + 1) * _CH, :] = d

    rs = (lax.broadcasted_iota(jnp.int32, (_NBLK, _E), 0) * _BLK).astype(jnp.float32)
    cb = jnp.broadcast_to(cum, (_NBLK, _E))
    be = jnp.sum((rs >= cb).astype(jnp.int32), axis=1, keepdims=True)
    sp_ref[0:_NBLK, :] = jnp.minimum(be, _E - 1)
    total = jnp.sum(pc, axis=1, keepdims=True)
    rs1 = (lax.broadcasted_iota(jnp.int32, (_NBLK, 1), 0) * _BLK).astype(jnp.float32)
    sp_ref[_NBLK:2 * _NBLK, :] = (rs1 < jnp.broadcast_to(total, (_NBLK, 1))).astype(jnp.int32)

    importance = jnp.sum(probs, axis=0, keepdims=True)
    aux_ref[...] = (jnp.sum(importance * counts, axis=1, keepdims=True)
                    * (_E / (_N * _N + 1e-6)))


def _prep(x, gamma, beta, Wg):
    return pl.pallas_call(
        _prep_body,
        out_shape=[
            jax.ShapeDtypeStruct((_N, _D), jnp.float32),
            jax.ShapeDtypeStruct((2 * _N, 1), jnp.int32),
            jax.ShapeDtypeStruct((2 * _N, 1), jnp.float32),
            jax.ShapeDtypeStruct((2 * _NBLK, 1), jnp.int32),
            jax.ShapeDtypeStruct((1, 1), jnp.float32),
        ],
    )(x, gamma.reshape(1, _D), beta.reshape(1, _D), Wg)


def _dispatch(xn, dest):
    chunk = 2 * _N // _NW
    mesh = plsc.VectorSubcoreMesh(core_axis_name="c", subcore_axis_name="s")

    @functools.partial(
        pl.kernel,
        out_type=jax.ShapeDtypeStruct((_NPAD, _D), jnp.float32),
        mesh=mesh,
        scratch_types=[
            pltpu.VMEM((chunk,), jnp.int32),
            pltpu.VMEM((chunk, _D), jnp.float32),
            pltpu.SemaphoreType.DMA,
        ],
    )
    def k(xn_hbm, dest_hbm, xg_hbm, idx_v, rows_v, sem):
        wid = lax.axis_index("s") * 2 + lax.axis_index("c")
        base = wid * chunk
        t0 = lax.rem(base, _N)
        pltpu.sync_copy(dest_hbm.at[pl.ds(base, chunk)], idx_v)
        pltpu.sync_copy(xn_hbm.at[pl.ds(t0, chunk)], rows_v)
        pltpu.async_copy(rows_v, xg_hbm.at[idx_v], sem).wait()

    return k(xn, dest)


def _ffn_body(sp_ref, xg_ref, w1_ref, b1_ref, w2_ref, b2_ref, og_ref):
    i = pl.program_id(0)

    @pl.when(sp_ref[_NBLK + i] == 1)
    def _():
        xb = xg_ref[...]
        h = lax.dot_general(xb, w1_ref[0], (((1,), (1,)), ((), ())),
                            preferred_element_type=jnp.float32) + b1_ref[0]
        g = 0.5 * h * (1.0 + lax.erf(h * 0.7071067811865476))
        out = lax.dot_general(g, w2_ref[0], (((1,), (1,)), ((), ())),
                              preferred_element_type=jnp.float32) + b2_ref[0]
        og_ref[...] = out


def _ffn(sp, xg, W1, b1, W2, b2):
    grid_spec = pltpu.PrefetchScalarGridSpec(
        num_scalar_prefetch=1,
        grid=(_NBLK,),
        in_specs=[
            pl.BlockSpec((_BLK, _D), lambda i, sp: (i, 0)),
            pl.BlockSpec((1, _F, _D), lambda i, sp: (sp[i], 0, 0)),
            pl.BlockSpec((1, 1, _F), lambda i, sp: (sp[i], 0, 0)),
            pl.BlockSpec((1, _D, _F), lambda i, sp: (sp[i], 0, 0)),
            pl.BlockSpec((1, 1, _D), lambda i, sp: (sp[i], 0, 0)),
        ],
        out_specs=pl.BlockSpec((_BLK, _D), lambda i, sp: (i, 0)),
    )
    return pl.pallas_call(
        _ffn_body,
        grid_spec=grid_spec,
        out_shape=jax.ShapeDtypeStruct((_NPAD, _D), jnp.float32),
    )(sp, xg, W1, b1.reshape(_E, 1, _F), W2, b2.reshape(_E, 1, _D))


def _combine(x, og, dest, tp):
    tpw = _N // _NW
    sub = 16
    nsub = tpw // sub
    mesh = plsc.VectorSubcoreMesh(core_axis_name="c", subcore_axis_name="s")

    @functools.partial(
        pl.kernel,
        out_type=jax.ShapeDtypeStruct((_N, _D), jnp.float32),
        mesh=mesh,
        scratch_types=[
            pltpu.VMEM((tpw,), jnp.int32),
            pltpu.VMEM((tpw,), jnp.int32),
            pltpu.VMEM((tpw + 16,), jnp.float32),
            pltpu.VMEM((tpw + 16,), jnp.float32),
            [pltpu.VMEM((sub, _D), jnp.float32)] * 2,
            [pltpu.VMEM((sub, _D), jnp.float32)] * 2,
            [pltpu.VMEM((sub, _D), jnp.float32)] * 2,
            [pltpu.VMEM((sub, _D), jnp.float32)] * 2,
            [pltpu.SemaphoreType.DMA] * 2,
            [pltpu.SemaphoreType.DMA] * 2,
        ],
    )
    def k(x_hbm, og_hbm, dest_hbm, tp_hbm, out_hbm,
          i0_v, i1_v, w0_v, w1_v, xb_v, g0_v, g1_v, ob_v, sem_in, sem_out):
        wid = lax.axis_index("s") * 2 + lax.axis_index("c")
        base = wid * tpw
        pltpu.sync_copy(dest_hbm.at[pl.ds(base, tpw)], i0_v)
        pltpu.sync_copy(dest_hbm.at[pl.ds(_N + base, tpw)], i1_v)
        pltpu.sync_copy(tp_hbm.at[pl.ds(base, tpw)], w0_v.at[pl.ds(0, tpw)])
        pltpu.sync_copy(tp_hbm.at[pl.ds(_N + base, tpw)], w1_v.at[pl.ds(0, tpw)])

        def issue_in(it):
            p = it % 2
            cps = [
                pltpu.make_async_copy(x_hbm.at[pl.ds(base + it * sub, sub)],
                                      xb_v[p], sem_in[p]),
                pltpu.make_async_copy(og_hbm.at[i0_v[pl.ds(it * sub, sub)]],
                                      g0_v[p], sem_in[p]),
                pltpu.make_async_copy(og_hbm.at[i1_v[pl.ds(it * sub, sub)]],
                                      g1_v[p], sem_in[p]),
            ]
            for cp in cps:
                cp.start()
            return cps

        inflight = {0: issue_in(0), 1: issue_in(1)}
        outflight = {}
        for it in range(nsub):
            p = it % 2
            for cp in inflight.pop(it):
                cp.wait()
            if it >= 2:
                outflight.pop(it - 2).wait()

            def per_token(t, _):
                a = w0_v[pl.ds(it * sub + t, 16)][0]
                b = w1_v[pl.ds(it * sub + t, 16)][0]
                for cc in range(_D // 16):
                    sl = pl.ds(cc * 16, 16)
                    ob_v[p][t, sl] = (xb_v[p][t, sl] + a * g0_v[p][t, sl]
                                      + b * g1_v[p][t, sl])
                return 0

            lax.fori_loop(0, sub, per_token, 0)
            ocp = pltpu.make_async_copy(
                ob_v[p], out_hbm.at[pl.ds(base + it * sub, sub)], sem_out[p])
            ocp.start()
            outflight[it] = ocp
            if it + 2 < nsub:
                inflight[it + 2] = issue_in(it + 2)
        for it in (nsub - 2, nsub - 1):
            outflight.pop(it).wait()

    return k(x, og, dest, tp)


def kernel(x, gamma, beta, Wg, W1, b1, W2, b2):
    xn, dest2, tp2, spv, aux = _prep(x, gamma, beta, Wg)
    dest = dest2.reshape(2 * _N)
    xg = _dispatch(xn, dest)
    og = _ffn(spv.reshape(2 * _NBLK), xg, W1, b1, W2, b2)
    final = _combine(x, og, dest, tp2.reshape(2 * _N))
    return final, aux[0, 0]


# speedup vs baseline: 1.3603x; 1.3603x over previous
import functools

import jax
import jax.numpy as jnp
from jax import lax
from jax.experimental import pallas as pl
from jax.experimental.pallas import tpu as pltpu
from jax.experimental.pallas import tpu_sc as plsc

_N = 2048
_D = 768
_E = 8
_F = 3072
_BLK = 256
_NBLK = 2 * _N // _BLK + _E
_NPAD = _NBLK * _BLK
_CH = 512
_NW = 32


def _prep_body(x_ref, g_ref, b_ref, wg_ref,
               xn_ref, dest_ref, tp_ref, sp_ref, aux_ref):
    xv = x_ref[...]
    mu = jnp.mean(xv, axis=1, keepdims=True)
    xc = xv - mu
    var = jnp.mean(xc * xc, axis=1, keepdims=True)
    xn = xc * lax.rsqrt(var + 1e-5) * g_ref[...] + b_ref[...]
    xn_ref[...] = xn

    logits = lax.dot_general(xn, wg_ref[...], (((1,), (1,)), ((), ())),
                             preferred_element_type=jnp.float32)
    z = logits - jnp.max(logits, axis=1, keepdims=True)
    ez = jnp.exp(z)
    probs = ez / jnp.sum(ez, axis=1, keepdims=True)

    idx = lax.broadcasted_iota(jnp.int32, (_N, _E), 1)
    m1 = jnp.max(probs, axis=1, keepdims=True)
    i1 = jnp.min(jnp.where(probs == m1, idx, _E), axis=1, keepdims=True)
    masked = jnp.where(idx == i1, -1.0, probs)
    m2 = jnp.max(masked, axis=1, keepdims=True)
    i2 = jnp.min(jnp.where(masked == m2, idx, _E), axis=1, keepdims=True)
    s = m1 + m2
    tp_ref[0:_N, :] = m1 / s
    tp_ref[_N:2 * _N, :] = m2 / s

    oh0 = (idx == i1).astype(jnp.float32)
    oh1 = (idx == i2).astype(jnp.float32)

    tri = (lax.broadcasted_iota(jnp.int32, (_CH, _CH), 0)
           > lax.broadcasted_iota(jnp.int32, (_CH, _CH), 1)).astype(jnp.float32)
    carry = jnp.zeros((1, _E), jnp.float32)
    ranks = []
    chunks = []
    for c in range(2 * _N // _CH):
        src = oh0 if c < _N // _CH else oh1
        r0 = (c % (_N // _CH)) * _CH
        ohc = src[r0:r0 + _CH, :]
        rm = lax.dot_general(tri, ohc, (((1,), (0,)), ((), ())),
                             preferred_element_type=jnp.float32) + carry
        ranks.append(jnp.sum(rm * ohc, axis=1, keepdims=True))
        chunks.append(ohc)
        carry = carry + jnp.sum(ohc, axis=0, keepdims=True)
    counts = carry

    pc = jnp.ceil(counts / _BLK) * _BLK
    lt = (lax.broadcasted_iota(jnp.int32, (_E, _E), 0)
          < lax.broadcasted_iota(jnp.int32, (_E, _E), 1)).astype(jnp.float32)
    le = (lax.broadcasted_iota(jnp.int32, (_E, _E), 0)
          <= lax.broadcasted_iota(jnp.int32, (_E, _E), 1)).astype(jnp.float32)
    po = lax.dot_general(pc, lt, (((1,), (0,)), ((), ())),
                         preferred_element_type=jnp.float32)
    cum = lax.dot_general(pc, le, (((1,), (0,)), ((), ())),
                          preferred_element_type=jnp.float32)

    for c in range(2 * _N // _CH):
        ohc = chunks[c]
        posel = jnp.sum(po * ohc, axis=1, keepdims=True)
        d = (ranks[c] + posel).astype(jnp.int32)
        dest_ref[c * _CH:(c + 1) * _CH, :] = d

    rs = (lax.broadcasted_iota(jnp.int32, (_NBLK, _E), 0) * _BLK).astype(jnp.float32)
    cb = jnp.broadcast_to(cum, (_NBLK, _E))
    be = jnp.sum((rs >= cb).astype(jnp.int32), axis=1, keepdims=True)
    sp_ref[0:_NBLK, :] = jnp.minimum(be, _E - 1)
    total = jnp.sum(pc, axis=1, keepdims=True)
    rs1 = (lax.broadcasted_iota(jnp.int32, (_NBLK, 1), 0) * _BLK).astype(jnp.float32)
    sp_ref[_NBLK:2 * _NBLK, :] = (rs1 < jnp.broadcast_to(total, (_NBLK, 1))).astype(jnp.int32)

    importance = jnp.sum(probs, axis=0, keepdims=True)
    aux_ref[...] = (jnp.sum(importance * counts, axis=1, keepdims=True)
                    * (_E / (_N * _N + 1e-6)))


def _prep(x, gamma, beta, Wg):
    return pl.pallas_call(
        _prep_body,
        out_shape=[
            jax.ShapeDtypeStruct((_N, _D), jnp.float32),
            jax.ShapeDtypeStruct((2 * _N, 1), jnp.int32),
            jax.ShapeDtypeStruct((2 * _N, 1), jnp.float32),
            jax.ShapeDtypeStruct((2 * _NBLK, 1), jnp.int32),
            jax.ShapeDtypeStruct((1, 1), jnp.float32),
        ],
    )(x, gamma.reshape(1, _D), beta.reshape(1, _D), Wg)


def _dispatch(xn, dest):
    chunk = 2 * _N // _NW
    mesh = plsc.VectorSubcoreMesh(core_axis_name="c", subcore_axis_name="s")

    @functools.partial(
        pl.kernel,
        out_type=jax.ShapeDtypeStruct((_NPAD, _D), jnp.float32),
        mesh=mesh,
        scratch_types=[
            pltpu.VMEM((chunk,), jnp.int32),
            pltpu.VMEM((chunk, _D), jnp.float32),
            pltpu.SemaphoreType.DMA,
        ],
    )
    def k(xn_hbm, dest_hbm, xg_hbm, idx_v, rows_v, sem):
        wid = lax.axis_index("s") * 2 + lax.axis_index("c")
        base = wid * chunk
        t0 = lax.rem(base, _N)
        pltpu.sync_copy(dest_hbm.at[pl.ds(base, chunk)], idx_v)
        pltpu.sync_copy(xn_hbm.at[pl.ds(t0, chunk)], rows_v)
        pltpu.async_copy(rows_v, xg_hbm.at[idx_v], sem).wait()

    return k(xn, dest)


def _ffn_body(sp_ref, xg_ref, w1_ref, b1_ref, w2_ref, b2_ref, og_ref):
    i = pl.program_id(0)

    @pl.when(sp_ref[_NBLK + i] == 1)
    def _():
        xb = xg_ref[...]
        h = lax.dot_general(xb, w1_ref[0], (((1,), (1,)), ((), ())),
                            preferred_element_type=jnp.float32) + b1_ref[0]
        g = 0.5 * h * (1.0 + lax.erf(h * 0.7071067811865476))
        out = lax.dot_general(g, w2_ref[0], (((1,), (1,)), ((), ())),
                              preferred_element_type=jnp.float32) + b2_ref[0]
        og_ref[...] = out


def _ffn(sp, xg, W1, b1, W2, b2):
    grid_spec = pltpu.PrefetchScalarGridSpec(
        num_scalar_prefetch=1,
        grid=(_NBLK,),
        in_specs=[
            pl.BlockSpec((_BLK, _D), lambda i, sp: (i, 0)),
            pl.BlockSpec((1, _F, _D), lambda i, sp: (sp[i], 0, 0)),
            pl.BlockSpec((1, 1, _F), lambda i, sp: (sp[i], 0, 0)),
            pl.BlockSpec((1, _D, _F), lambda i, sp: (sp[i], 0, 0)),
            pl.BlockSpec((1, 1, _D), lambda i, sp: (sp[i], 0, 0)),
        ],
        out_specs=pl.BlockSpec((_BLK, _D), lambda i, sp: (i, 0)),
    )
    return pl.pallas_call(
        _ffn_body,
        grid_spec=grid_spec,
        out_shape=jax.ShapeDtypeStruct((_NPAD, _D), jnp.float32),
    )(sp, xg, W1, b1.reshape(_E, 1, _F), W2, b2.reshape(_E, 1, _D))


def _combine(x, og, dest, tp):
    tpw = _N // _NW
    sub = 16
    nsub = tpw // sub
    mesh = plsc.VectorSubcoreMesh(core_axis_name="c", subcore_axis_name="s")

    @functools.partial(
        pl.kernel,
        out_type=jax.ShapeDtypeStruct((_N, _D), jnp.float32),
        mesh=mesh,
        scratch_types=[
            pltpu.VMEM((tpw,), jnp.int32),
            pltpu.VMEM((tpw,), jnp.int32),
            pltpu.VMEM((tpw + 16,), jnp.float32),
            pltpu.VMEM((tpw + 16,), jnp.float32),
            [pltpu.VMEM((sub, _D), jnp.float32)] * 2,
            [pltpu.VMEM((sub, _D), jnp.float32)] * 2,
            [pltpu.VMEM((sub, _D), jnp.float32)] * 2,
            [pltpu.VMEM((sub, _D), jnp.float32)] * 2,
            [pltpu.SemaphoreType.DMA] * 2,
            [pltpu.SemaphoreType.DMA] * 2,
        ],
    )
    def k(x_hbm, og_hbm, dest_hbm, tp_hbm, out_hbm,
          i0_v, i1_v, w0_v, w1_v, xb_v, g0_v, g1_v, ob_v, sem_in, sem_out):
        wid = lax.axis_index("s") * 2 + lax.axis_index("c")
        base = wid * tpw
        pltpu.sync_copy(dest_hbm.at[pl.ds(base, tpw)], i0_v)
        pltpu.sync_copy(dest_hbm.at[pl.ds(_N + base, tpw)], i1_v)
        pltpu.sync_copy(tp_hbm.at[pl.ds(base, tpw)], w0_v.at[pl.ds(0, tpw)])
        pltpu.sync_copy(tp_hbm.at[pl.ds(_N + base, tpw)], w1_v.at[pl.ds(0, tpw)])

        def issue_in(it):
            p = it % 2
            cps = [
                pltpu.make_async_copy(x_hbm.at[pl.ds(base + it * sub, sub)],
                                      xb_v[p], sem_in[p]),
                pltpu.make_async_copy(og_hbm.at[i0_v[pl.ds(it * sub, sub)]],
                                      g0_v[p], sem_in[p]),
                pltpu.make_async_copy(og_hbm.at[i1_v[pl.ds(it * sub, sub)]],
                                      g1_v[p], sem_in[p]),
            ]
            for cp in cps:
                cp.start()
            return cps

        inflight = {0: issue_in(0), 1: issue_in(1)}
        outflight = {}
        for it in range(nsub):
            p = it % 2
            for cp in inflight.pop(it):
                cp.wait()
            if it >= 2:
                outflight.pop(it - 2).wait()

            def per_token(t, _):
                a = w0_v[pl.ds(it * sub + t, 16)][0]
                b = w1_v[pl.ds(it * sub + t, 16)][0]
                for cc in range(_D // 16):
                    sl = pl.ds(cc * 16, 16)
                    ob_v[p][t, sl] = (xb_v[p][t, sl] + a * g0_v[p][t, sl]
                                      + b * g1_v[p][t, sl])
                return 0

            lax.fori_loop(0, sub, per_token, 0)
            ocp = pltpu.make_async_copy(
                ob_v[p], out_hbm.at[pl.ds(base + it * sub, sub)], sem_out[p])
            ocp.start()
            outflight[it] = ocp
            if it + 2 < nsub:
                inflight[it + 2] = issue_in(it + 2)
        for it in (nsub - 2, nsub - 1):
            outflight.pop(it).wait()

    return k(x, og, dest, tp)


def kernel(x, gamma, beta, Wg, W1, b1, W2, b2):
    xn, dest2, tp2, spv, aux = _prep(x, gamma, beta, Wg)
    dest = dest2.reshape(2 * _N)
    xg = _dispatch(xn, dest)
    og = _ffn(spv.reshape(2 * _NBLK), xg, W1, b1, W2, b2)
    final = _combine(x, og, dest, tp2.reshape(2 * _N))
    return final, aux[0, 0]


# speedup vs baseline: 1.3706x; 1.0076x over previous
import functools

import jax
import jax.numpy as jnp
from jax import lax
from jax.experimental import pallas as pl
from jax.experimental.pallas import tpu as pltpu
from jax.experimental.pallas import tpu_sc as plsc

_N = 2048
_D = 768
_E = 8
_F = 3072
_BLK = 256
_NBLK = 2 * _N // _BLK + _E
_NPAD = _NBLK * _BLK
_CH = 512
_NW = 32


def _prep_body(x_ref, g_ref, b_ref, wg_ref,
               xn_ref, dest_ref, tp_ref, sp_ref, aux_ref):
    xv = x_ref[...]
    mu = jnp.mean(xv, axis=1, keepdims=True)
    xc = xv - mu
    var = jnp.mean(xc * xc, axis=1, keepdims=True)
    xn = xc * lax.rsqrt(var + 1e-5) * g_ref[...] + b_ref[...]
    xn_ref[...] = xn

    logits = lax.dot_general(xn, wg_ref[...], (((1,), (1,)), ((), ())),
                             preferred_element_type=jnp.float32)
    z = logits - jnp.max(logits, axis=1, keepdims=True)
    ez = jnp.exp(z)
    probs = ez / jnp.sum(ez, axis=1, keepdims=True)

    idx = lax.broadcasted_iota(jnp.int32, (_N, _E), 1)
    m1 = jnp.max(probs, axis=1, keepdims=True)
    i1 = jnp.min(jnp.where(probs == m1, idx, _E), axis=1, keepdims=True)
    masked = jnp.where(idx == i1, -1.0, probs)
    m2 = jnp.max(masked, axis=1, keepdims=True)
    i2 = jnp.min(jnp.where(masked == m2, idx, _E), axis=1, keepdims=True)
    s = m1 + m2
    tp_ref[0:_N, :] = m1 / s
    tp_ref[_N:2 * _N, :] = m2 / s

    oh0 = (idx == i1).astype(jnp.float32)
    oh1 = (idx == i2).astype(jnp.float32)

    tri = (lax.broadcasted_iota(jnp.int32, (_CH, _CH), 0)
           > lax.broadcasted_iota(jnp.int32, (_CH, _CH), 1)).astype(jnp.float32)
    carry = jnp.zeros((1, _E), jnp.float32)
    ranks = []
    chunks = []
    for c in range(2 * _N // _CH):
        src = oh0 if c < _N // _CH else oh1
        r0 = (c % (_N // _CH)) * _CH
        ohc = src[r0:r0 + _CH, :]
        rm = lax.dot_general(tri, ohc, (((1,), (0,)), ((), ())),
                             preferred_element_type=jnp.float32) + carry
        ranks.append(jnp.sum(rm * ohc, axis=1, keepdims=True))
        chunks.append(ohc)
        carry = carry + jnp.sum(ohc, axis=0, keepdims=True)
    counts = carry

    pc = jnp.ceil(counts / _BLK) * _BLK
    lt = (lax.broadcasted_iota(jnp.int32, (_E, _E), 0)
          < lax.broadcasted_iota(jnp.int32, (_E, _E), 1)).astype(jnp.float32)
    le = (lax.broadcasted_iota(jnp.int32, (_E, _E), 0)
          <= lax.broadcasted_iota(jnp.int32, (_E, _E), 1)).astype(jnp.float32)
    po = lax.dot_general(pc, lt, (((1,), (0,)), ((), ())),
                         preferred_element_type=jnp.float32)
    cum = lax.dot_general(pc, le, (((1,), (0,)), ((), ())),
                          preferred_element_type=jnp.float32)

    for c in range(2 * _N // _CH):
        ohc = chunks[c]
        posel = jnp.sum(po * ohc, axis=1, keepdims=True)
        d = (ranks[c] + posel).astype(jnp.int32)
        dest_ref[c * _CH:(c + 1) * _CH, :] = d

    rs = (lax.broadcasted_iota(jnp.int32, (_NBLK, _E), 0) * _BLK).astype(jnp.float32)
    cb = jnp.broadcast_to(cum, (_NBLK, _E))
    be = jnp.sum((rs >= cb).astype(jnp.int32), axis=1, keepdims=True)
    sp_ref[0:_NBLK, :] = jnp.minimum(be, _E - 1)
    total = jnp.sum(pc, axis=1, keepdims=True)
    rs1 = (lax.broadcasted_iota(jnp.int32, (_NBLK, 1), 0) * _BLK).astype(jnp.float32)
    sp_ref[_NBLK:2 * _NBLK, :] = (rs1 < jnp.broadcast_to(total, (_NBLK, 1))).astype(jnp.int32)

    importance = jnp.sum(probs, axis=0, keepdims=True)
    aux_ref[...] = (jnp.sum(importance * counts, axis=1, keepdims=True)
                    * (_E / (_N * _N + 1e-6)))


def _prep(x, gamma, beta, Wg):
    return pl.pallas_call(
        _prep_body,
        out_shape=[
            jax.ShapeDtypeStruct((_N, _D), jnp.float32),
            jax.ShapeDtypeStruct((2 * _N, 1), jnp.int32),
            jax.ShapeDtypeStruct((2 * _N, 1), jnp.float32),
            jax.ShapeDtypeStruct((2 * _NBLK, 1), jnp.int32),
            jax.ShapeDtypeStruct((1, 1), jnp.float32),
        ],
    )(x, gamma.reshape(1, _D), beta.reshape(1, _D), Wg)


def _dispatch(xn, dest):
    chunk = 2 * _N // _NW
    mesh = plsc.VectorSubcoreMesh(core_axis_name="c", subcore_axis_name="s")

    half = chunk // 2

    @functools.partial(
        pl.kernel,
        out_type=jax.ShapeDtypeStruct((_NPAD, _D), jnp.float32),
        mesh=mesh,
        scratch_types=[
            [pltpu.VMEM((half,), jnp.int32)] * 2,
            [pltpu.VMEM((half, _D), jnp.float32)] * 2,
            pltpu.SemaphoreType.DMA,
            pltpu.SemaphoreType.DMA,
        ],
    )
    def k(xn_hbm, dest_hbm, xg_hbm, idx_v, rows_v, sem_in, sem_sc):
        wid = lax.axis_index("s") * 2 + lax.axis_index("c")
        base = wid * chunk
        t0 = lax.rem(base, _N)
        fetches = []
        for p in range(2):
            cps = [
                pltpu.make_async_copy(
                    dest_hbm.at[pl.ds(base + p * half, half)], idx_v[p], sem_in),
                pltpu.make_async_copy(
                    xn_hbm.at[pl.ds(t0 + p * half, half)], rows_v[p], sem_in),
            ]
            for cp in cps:
                cp.start()
            fetches.append(cps)
        scs = []
        for p in range(2):
            for cp in fetches[p]:
                cp.wait()
            sc = pltpu.make_async_copy(rows_v[p], xg_hbm.at[idx_v[p]], sem_sc)
            sc.start()
            scs.append(sc)
        for sc in scs:
            sc.wait()

    return k(xn, dest)


def _ffn_body(sp_ref, xg_ref, w1_ref, b1_ref, w2_ref, b2_ref, og_ref):
    i = pl.program_id(0)

    @pl.when(sp_ref[_NBLK + i] == 1)
    def _():
        xb = xg_ref[...]
        h = lax.dot_general(xb, w1_ref[0], (((1,), (1,)), ((), ())),
                            preferred_element_type=jnp.float32) + b1_ref[0]
        g = 0.5 * h * (1.0 + lax.erf(h * 0.7071067811865476))
        out = lax.dot_general(g, w2_ref[0], (((1,), (1,)), ((), ())),
                              preferred_element_type=jnp.float32) + b2_ref[0]
        og_ref[...] = out


def _ffn(sp, xg, W1, b1, W2, b2):
    grid_spec = pltpu.PrefetchScalarGridSpec(
        num_scalar_prefetch=1,
        grid=(_NBLK,),
        in_specs=[
            pl.BlockSpec((_BLK, _D), lambda i, sp: (i, 0)),
            pl.BlockSpec((1, _F, _D), lambda i, sp: (sp[i], 0, 0)),
            pl.BlockSpec((1, 1, _F), lambda i, sp: (sp[i], 0, 0)),
            pl.BlockSpec((1, _D, _F), lambda i, sp: (sp[i], 0, 0)),
            pl.BlockSpec((1, 1, _D), lambda i, sp: (sp[i], 0, 0)),
        ],
        out_specs=pl.BlockSpec((_BLK, _D), lambda i, sp: (i, 0)),
    )
    return pl.pallas_call(
        _ffn_body,
        grid_spec=grid_spec,
        out_shape=jax.ShapeDtypeStruct((_NPAD, _D), jnp.float32),
    )(sp, xg, W1, b1.reshape(_E, 1, _F), W2, b2.reshape(_E, 1, _D))


def _combine(x, og, dest, tp):
    tpw = _N // _NW
    sub = 16
    nsub = tpw // sub
    mesh = plsc.VectorSubcoreMesh(core_axis_name="c", subcore_axis_name="s")

    @functools.partial(
        pl.kernel,
        out_type=jax.ShapeDtypeStruct((_N, _D), jnp.float32),
        mesh=mesh,
        scratch_types=[
            pltpu.VMEM((tpw,), jnp.int32),
            pltpu.VMEM((tpw,), jnp.int32),
            pltpu.VMEM((tpw + 16,), jnp.float32),
            pltpu.VMEM((tpw + 16,), jnp.float32),
            [pltpu.VMEM((sub, _D), jnp.float32)] * 2,
            [pltpu.VMEM((sub, _D), jnp.float32)] * 2,
            [pltpu.VMEM((sub, _D), jnp.float32)] * 2,
            [pltpu.VMEM((sub, _D), jnp.float32)] * 2,
            [pltpu.SemaphoreType.DMA] * 2,
            [pltpu.SemaphoreType.DMA] * 2,
        ],
    )
    def k(x_hbm, og_hbm, dest_hbm, tp_hbm, out_hbm,
          i0_v, i1_v, w0_v, w1_v, xb_v, g0_v, g1_v, ob_v, sem_in, sem_out):
        wid = lax.axis_index("s") * 2 + lax.axis_index("c")
        base = wid * tpw
        pre = [
            pltpu.make_async_copy(dest_hbm.at[pl.ds(base, tpw)], i0_v,
                                  sem_out[0]),
            pltpu.make_async_copy(dest_hbm.at[pl.ds(_N + base, tpw)], i1_v,
                                  sem_out[0]),
            pltpu.make_async_copy(tp_hbm.at[pl.ds(base, tpw)],
                                  w0_v.at[pl.ds(0, tpw)], sem_out[1]),
            pltpu.make_async_copy(tp_hbm.at[pl.ds(_N + base, tpw)],
                                  w1_v.at[pl.ds(0, tpw)], sem_out[1]),
        ]
        for cp in pre:
            cp.start()
        for cp in pre:
            cp.wait()

        def issue_in(it):
            p = it % 2
            cps = [
                pltpu.make_async_copy(x_hbm.at[pl.ds(base + it * sub, sub)],
                                      xb_v[p], sem_in[p]),
                pltpu.make_async_copy(og_hbm.at[i0_v[pl.ds(it * sub, sub)]],
                                      g0_v[p], sem_in[p]),
                pltpu.make_async_copy(og_hbm.at[i1_v[pl.ds(it * sub, sub)]],
                                      g1_v[p], sem_in[p]),
            ]
            for cp in cps:
                cp.start()
            return cps

        inflight = {0: issue_in(0), 1: issue_in(1)}
        outflight = {}
        for it in range(nsub):
            p = it % 2
            for cp in inflight.pop(it):
                cp.wait()
            if it >= 2:
                outflight.pop(it - 2).wait()

            def per_token(t, _):
                a = w0_v[pl.ds(it * sub + t, 16)][0]
                b = w1_v[pl.ds(it * sub + t, 16)][0]
                for cc in range(_D // 16):
                    sl = pl.ds(cc * 16, 16)
                    ob_v[p][t, sl] = (xb_v[p][t, sl] + a * g0_v[p][t, sl]
                                      + b * g1_v[p][t, sl])
                return 0

            lax.fori_loop(0, sub, per_token, 0)
            ocp = pltpu.make_async_copy(
                ob_v[p], out_hbm.at[pl.ds(base + it * sub, sub)], sem_out[p])
            ocp.start()
            outflight[it] = ocp
            if it + 2 < nsub:
                inflight[it + 2] = issue_in(it + 2)
        for it in (nsub - 2, nsub - 1):
            outflight.pop(it).wait()

    return k(x, og, dest, tp)


def kernel(x, gamma, beta, Wg, W1, b1, W2, b2):
    xn, dest2, tp2, spv, aux = _prep(x, gamma, beta, Wg)
    dest = dest2.reshape(2 * _N)
    xg = _dispatch(xn, dest)
    og = _ffn(spv.reshape(2 * _NBLK), xg, W1, b1, W2, b2)
    final = _combine(x, og, dest, tp2.reshape(2 * _N))
    return final, aux[0, 0]
